# baseline (speedup 1.0000x reference)
import jax
import jax.numpy as jnp
from jax import lax
from jax.experimental import pallas as pl
from jax.experimental.pallas import tpu as pltpu
from jax.experimental.pallas import tpu_sc as plsc

N = 10000
E = 320000
D = 128
NC = 2
NS = 16
NW = NC * NS
EPT = E // NW
CH = 100
NCHP = 100
EPTP = NCHP * CH
NP = 10240
RT = NP // NS
GCH = 80
ZB = 64

_mesh = plsc.VectorSubcoreMesh(core_axis_name="c", subcore_axis_name="s")


def _zero_rows(buf, nrows, width):
    z16 = jnp.zeros((16,), jnp.float32)

    def _fill(i, _):
        for j in range(width // 16):
            buf[i, pl.ds(j * 16, 16)] = z16
        return 0
    lax.fori_loop(0, nrows, _fill, 0)


def _sc_prelim_body(t1_hbm, label_hbm, dst_hbm, z0_hbm, deg_hbm,
                    label_v, rows_v, dst_v, ones_v, zero_v, degbuf, sem):
    c = lax.axis_index("c")
    s = lax.axis_index("s")
    wid = c * NS + s

    _zero_rows(zero_v, ZB, D)
    for r in range(RT // ZB):
        pltpu.sync_copy(zero_v, degbuf.at[pl.ds(s * RT + r * ZB, ZB)])
    plsc.subcore_barrier()

    pltpu.sync_copy(label_hbm.at[wid], label_v)
    for g in range(4):
        pltpu.async_copy(t1_hbm.at[label_v.at[g]], rows_v, sem).wait()
        pltpu.sync_copy(rows_v, z0_hbm.at[pl.ds(wid * 320 + g * GCH, GCH)])

    pltpu.sync_copy(dst_hbm.at[wid], dst_v)
    o16 = jnp.ones((16,), jnp.float32)

    def _fill_ones(i, _):
        for j in range(D // 16):
            ones_v[i, pl.ds(j * 16, 16)] = o16
        return 0
    lax.fori_loop(0, CH, _fill_ones, 0)

    def _deg_chunk(g, _):
        pltpu.sync_copy(ones_v, degbuf.at[dst_v.at[g]], add=True)
        return 0
    lax.fori_loop(0, NCHP, _deg_chunk, 0)

    plsc.subcore_barrier()
    pltpu.sync_copy(degbuf.at[pl.ds(s * RT, RT)],
                    deg_hbm.at[c, pl.ds(s * RT, RT)])


_sc_prelim = pl.kernel(
    _sc_prelim_body,
    out_type=(jax.ShapeDtypeStruct((NP, D), jnp.float32),
              jax.ShapeDtypeStruct((NC, NP, D), jnp.float32)),
    mesh=_mesh,
    scratch_types=[
        pltpu.VMEM((4, GCH), jnp.int32),
        pltpu.VMEM((GCH, D), jnp.float32),
        pltpu.VMEM((NCHP, CH), jnp.int32),
        pltpu.VMEM((CH, D), jnp.float32),
        pltpu.VMEM((ZB, D), jnp.float32),
        pltpu.VMEM_SHARED((NP, D), jnp.float32),
        pltpu.SemaphoreType.DMA,
    ],
)


def _sc_edge_body(z_hbm, src_hbm, dst_hbm, out_hbm,
                  src_v, dst_v, rows_v, zero_v, agg, sem):
    c = lax.axis_index("c")
    s = lax.axis_index("s")
    wid = c * NS + s

    _zero_rows(zero_v, ZB, D)
    for r in range(RT // ZB):
        pltpu.sync_copy(zero_v, agg.at[pl.ds(s * RT + r * ZB, ZB)])

    pltpu.sync_copy(src_hbm.at[wid], src_v)
    pltpu.sync_copy(dst_hbm.at[wid], dst_v)
    plsc.subcore_barrier()

    def chunk(g, _):
        pltpu.async_copy(z_hbm.at[src_v.at[g]], rows_v, sem).wait()
        pltpu.sync_copy(rows_v, agg.at[dst_v.at[g]], add=True)
        return 0
    lax.fori_loop(0, NCHP, chunk, 0)

    plsc.subcore_barrier()
    pltpu.sync_copy(agg.at[pl.ds(s * RT, RT)],
                    out_hbm.at[c, pl.ds(s * RT, RT)])


_sc_edge = pl.kernel(
    _sc_edge_body,
    out_type=jax.ShapeDtypeStruct((NC, NP, D), jnp.float32),
    mesh=_mesh,
    scratch_types=[
        pltpu.VMEM((NCHP, CH), jnp.int32),
        pltpu.VMEM((NCHP, CH), jnp.int32),
        pltpu.VMEM((CH, D), jnp.float32),
        pltpu.VMEM((ZB, D), jnp.float32),
        pltpu.VMEM_SHARED((NP, D), jnp.float32),
        pltpu.SemaphoreType.DMA,
    ],
)


def _mm_body(a_ref, w_ref, o_ref):
    o_ref[...] = jnp.dot(a_ref[...], w_ref[...],
                         preferred_element_type=jnp.float32)


def _small_matmul(a, w):
    return pl.pallas_call(
        _mm_body,
        out_shape=jax.ShapeDtypeStruct((a.shape[0], w.shape[1]), jnp.float32),
    )(a, w)


def _tc_layer(aggp, degp, b, res, w):
    has_res = res is not None
    has_w = w is not None

    def body(*refs):
        agg_ref, deg_ref, b_ref = refs[0], refs[1], refs[2]
        i = 3
        res_ref = None
        w_ref = None
        if has_res:
            res_ref = refs[i]; i += 1
        if has_w:
            w_ref = refs[i]; i += 1
        h_ref = refs[i]; i += 1
        a = agg_ref[0] + agg_ref[1]
        deg = deg_ref[0, :, 0:1] + deg_ref[1, :, 0:1]
        h = jnp.maximum(a / jnp.clip(deg, 1.0, None) + b_ref[...], 0.0)
        if has_res:
            h = h + res_ref[...]
        h_ref[...] = h
        if has_w:
            z_ref = refs[i]
            z_ref[...] = jnp.dot(h, w_ref[...],
                                 preferred_element_type=jnp.float32)

    outs = [jax.ShapeDtypeStruct((NP, D), jnp.float32)]
    if has_w:
        outs.append(jax.ShapeDtypeStruct((NP, D), jnp.float32))
    args = [aggp, degp, b]
    if has_res:
        args.append(res)
    if has_w:
        args.append(w)
    r = pl.pallas_call(body, out_shape=tuple(outs))(*args)
    return (r[0], r[1]) if has_w else (r[0], None)


def kernel(label, edge_index, emb, W1, b1, W2, b2, W3, b3, W4, b4, W5, b5):
    label = label.astype(jnp.int32)
    pad = EPTP - EPT
    srcp = jnp.pad(edge_index[0].astype(jnp.int32).reshape(NW, EPT),
                   ((0, 0), (0, pad))).reshape(NW, NCHP, CH)
    dstp = jnp.pad(edge_index[1].astype(jnp.int32).reshape(NW, EPT),
                   ((0, 0), (0, pad)),
                   constant_values=NP - 1).reshape(NW, NCHP, CH)
    label3 = jnp.pad(label, (0, NP - N)).reshape(NW, 4, GCH)

    t1 = _small_matmul(emb, W1)
    z, degp_full = _sc_prelim(t1, label3, dstp)
    degp = degp_full[:, :, :8]

    bs = [b1.reshape(1, D), b2.reshape(1, D), b3.reshape(1, D),
          b4.reshape(1, D), b5.reshape(1, D)]
    ws_next = [W2, W3, W4, W5, None]

    h_prev = None
    h = None
    for layer in range(5):
        aggp = _sc_edge(z, srcp, dstp)
        res = h_prev if layer in (1, 3) else None
        h, z = _tc_layer(aggp, degp, bs[layer], res, ws_next[layer])
        if layer in (0, 2):
            h_prev = h
        else:
            h_prev = None
    return h[:N]


# speedup vs baseline: 1.1692x; 1.1692x over previous
import jax
import jax.numpy as jnp
from jax import lax
from jax.experimental import pallas as pl
from jax.experimental.pallas import tpu as pltpu
from jax.experimental.pallas import tpu_sc as plsc

N = 10000
E = 320000
D = 128
NC = 2
NS = 16
NW = NC * NS
EPT = E // NW
CH = 100
NCHP = 100
EPTP = NCHP * CH
NP = 10240
RT = NP // NS
GCH = 80
ZB = 64

_mesh = plsc.VectorSubcoreMesh(core_axis_name="c", subcore_axis_name="s")


def _zero_rows(buf, nrows, width):
    z16 = jnp.zeros((16,), jnp.float32)

    def _fill(i, _):
        for j in range(width // 16):
            buf[i, pl.ds(j * 16, 16)] = z16
        return 0
    lax.fori_loop(0, nrows, _fill, 0)


def _sc_prelim_body(t1_hbm, label_hbm, dst_hbm, z0_hbm, deg_hbm,
                    label_v, rows_v, dst_v, ones_v, zero_v, degbuf, sem):
    c = lax.axis_index("c")
    s = lax.axis_index("s")
    wid = c * NS + s

    _zero_rows(zero_v, ZB, D)
    for r in range(RT // ZB):
        pltpu.sync_copy(zero_v, degbuf.at[pl.ds(s * RT + r * ZB, ZB)])
    plsc.subcore_barrier()

    pltpu.sync_copy(label_hbm.at[wid], label_v)
    for g in range(4):
        pltpu.async_copy(t1_hbm.at[label_v.at[g]], rows_v, sem).wait()
        pltpu.sync_copy(rows_v, z0_hbm.at[pl.ds(wid * 320 + g * GCH, GCH)])

    pltpu.sync_copy(dst_hbm.at[wid], dst_v)
    o16 = jnp.ones((16,), jnp.float32)

    def _fill_ones(i, _):
        for j in range(D // 16):
            ones_v[i, pl.ds(j * 16, 16)] = o16
        return 0
    lax.fori_loop(0, CH, _fill_ones, 0)

    def _deg_chunk(g, _):
        pltpu.sync_copy(ones_v, degbuf.at[dst_v.at[g]], add=True)
        return 0
    lax.fori_loop(0, NCHP, _deg_chunk, 0)

    plsc.subcore_barrier()
    pltpu.sync_copy(degbuf.at[pl.ds(s * RT, RT)],
                    deg_hbm.at[c, pl.ds(s * RT, RT)])


_sc_prelim = pl.kernel(
    _sc_prelim_body,
    out_type=(jax.ShapeDtypeStruct((NP, D), jnp.float32),
              jax.ShapeDtypeStruct((NC, NP, D), jnp.float32)),
    mesh=_mesh,
    scratch_types=[
        pltpu.VMEM((4, GCH), jnp.int32),
        pltpu.VMEM((GCH, D), jnp.float32),
        pltpu.VMEM((NCHP, CH), jnp.int32),
        pltpu.VMEM((CH, D), jnp.float32),
        pltpu.VMEM((ZB, D), jnp.float32),
        pltpu.VMEM_SHARED((NP, D), jnp.float32),
        pltpu.SemaphoreType.DMA,
    ],
)


def _sc_edge_body(z_hbm, src_hbm, dst_hbm, zeros_hbm, out_hbm,
                  src_v, dba, dbb, rows_a, rows_b, agg,
                  sem_da, sem_db, sem_a, sem_b):
    c = lax.axis_index("c")
    s = lax.axis_index("s")
    wid = c * NS + s

    pltpu.sync_copy(zeros_hbm, agg.at[pl.ds(s * RT, RT)])
    pltpu.sync_copy(src_hbm.at[wid], src_v)
    plsc.subcore_barrier()

    def pair(i, _):
        a = 2 * i
        da = pltpu.async_copy(dst_hbm.at[wid, a], dba.at[0], sem_da)
        db = pltpu.async_copy(dst_hbm.at[wid, a + 1], dbb.at[0], sem_db)
        ga = pltpu.async_copy(z_hbm.at[src_v.at[a]], rows_a, sem_a)
        gb = pltpu.async_copy(z_hbm.at[src_v.at[a + 1]], rows_b, sem_b)
        ga.wait()
        da.wait()
        pltpu.sync_copy(rows_a, agg.at[dba.at[0]], add=True)
        gb.wait()
        db.wait()
        pltpu.sync_copy(rows_b, agg.at[dbb.at[0]], add=True)
        return 0
    lax.fori_loop(0, NCHP // 2, pair, 0)

    plsc.subcore_barrier()
    pltpu.sync_copy(agg.at[pl.ds(s * RT, RT)],
                    out_hbm.at[c, pl.ds(s * RT, RT)])


_sc_edge = pl.kernel(
    _sc_edge_body,
    out_type=jax.ShapeDtypeStruct((NC, NP, D), jnp.float32),
    mesh=_mesh,
    scratch_types=[
        pltpu.VMEM((NCHP, CH), jnp.int32),
        pltpu.VMEM((1, CH), jnp.int32),
        pltpu.VMEM((1, CH), jnp.int32),
        pltpu.VMEM((CH, D), jnp.float32),
        pltpu.VMEM((CH, D), jnp.float32),
        pltpu.VMEM_SHARED((NP, D), jnp.float32),
        pltpu.SemaphoreType.DMA,
        pltpu.SemaphoreType.DMA,
        pltpu.SemaphoreType.DMA,
        pltpu.SemaphoreType.DMA,
    ],
)


def _mm_body(a_ref, w_ref, o_ref):
    o_ref[...] = jnp.dot(a_ref[...], w_ref[...],
                         preferred_element_type=jnp.float32)


def _small_matmul(a, w):
    return pl.pallas_call(
        _mm_body,
        out_shape=jax.ShapeDtypeStruct((a.shape[0], w.shape[1]), jnp.float32),
    )(a, w)


def _tc_layer(aggp, degp, b, res, w):
    has_res = res is not None
    has_w = w is not None

    def body(*refs):
        agg_ref, deg_ref, b_ref = refs[0], refs[1], refs[2]
        i = 3
        res_ref = None
        w_ref = None
        if has_res:
            res_ref = refs[i]; i += 1
        if has_w:
            w_ref = refs[i]; i += 1
        h_ref = refs[i]; i += 1
        a = agg_ref[0] + agg_ref[1]
        deg = deg_ref[0, :, 0:1] + deg_ref[1, :, 0:1]
        h = jnp.maximum(a / jnp.clip(deg, 1.0, None) + b_ref[...], 0.0)
        if has_res:
            h = h + res_ref[...]
        h_ref[...] = h
        if has_w:
            z_ref = refs[i]
            z_ref[...] = jnp.dot(h, w_ref[...],
                                 preferred_element_type=jnp.float32)

    outs = [jax.ShapeDtypeStruct((NP, D), jnp.float32)]
    if has_w:
        outs.append(jax.ShapeDtypeStruct((NP, D), jnp.float32))
    args = [aggp, degp, b]
    if has_res:
        args.append(res)
    if has_w:
        args.append(w)
    r = pl.pallas_call(body, out_shape=tuple(outs))(*args)
    return (r[0], r[1]) if has_w else (r[0], None)


def kernel(label, edge_index, emb, W1, b1, W2, b2, W3, b3, W4, b4, W5, b5):
    label = label.astype(jnp.int32)
    pad = EPTP - EPT
    srcp = jnp.pad(edge_index[0].astype(jnp.int32).reshape(NW, EPT),
                   ((0, 0), (0, pad))).reshape(NW, NCHP, CH)
    dstp = jnp.pad(edge_index[1].astype(jnp.int32).reshape(NW, EPT),
                   ((0, 0), (0, pad)),
                   constant_values=NP - 1).reshape(NW, NCHP, CH)
    label3 = jnp.pad(label, (0, NP - N)).reshape(NW, 4, GCH)

    t1 = _small_matmul(emb, W1)
    z, degp_full = _sc_prelim(t1, label3, dstp)
    degp = degp_full[:, :, :8]

    bs = [b1.reshape(1, D), b2.reshape(1, D), b3.reshape(1, D),
          b4.reshape(1, D), b5.reshape(1, D)]
    ws_next = [W2, W3, W4, W5, None]

    zrows = jnp.zeros((RT, D), jnp.float32)
    h_prev = None
    h = None
    for layer in range(5):
        aggp = _sc_edge(z, srcp, dstp, zrows)
        res = h_prev if layer in (1, 3) else None
        h, z = _tc_layer(aggp, degp, bs[layer], res, ws_next[layer])
        if layer in (0, 2):
            h_prev = h
        else:
            h_prev = None
    return h[:N]
